# Initial kernel scaffold; baseline (speedup 1.0000x reference)
import jax
import jax.numpy as jnp
from jax.experimental import pallas as pl


def kernel(X, plus, cross, dec, psi, phi, snr, tensors, vertices, psd, idx, offsets):
    raise NotImplementedError("write your pallas kernel here")


# speedup vs baseline: 1.4615x; 1.4615x over previous
import functools

import jax
import jax.numpy as jnp
import numpy as np
from jax.experimental import pallas as pl
from jax.experimental.pallas import tpu as pltpu

SAMPLE_RATE = 2048.0
W = 4096
KS = 2048
NW = 4096
B = 256
NI = 2
W2 = W + KS
C_LIGHT = 299792458.0
F0 = 64
NF = 2048
NBIN_MAX = 2048

_t = np.arange(W)
_f = np.arange(F0, F0 + NF)
_m = (_t[:, None] * _f[None, :]) % W
_ang = -2.0 * np.pi * _m / W
_COS = np.cos(_ang).astype(np.float32)
_SIN = np.sin(_ang).astype(np.float32)
del _t, _f, _m, _ang


def _gather_body(idx_ref, plus_ref, cross_ref, hp2_ref, hc2_ref):
    row_p = plus_ref[0, 0, :]
    row_c = cross_ref[0, 0, :]
    hp2_ref[0, 0, :W] = row_p
    hp2_ref[0, 0, W:] = row_p[:KS]
    hc2_ref[0, 0, :W] = row_c
    hc2_ref[0, 0, W:] = row_c[:KS]


def _gather_rows(idx, plus, cross):
    grid_spec = pltpu.PrefetchScalarGridSpec(
        num_scalar_prefetch=1,
        grid=(B,),
        in_specs=[
            pl.BlockSpec((1, 1, W), lambda b, idx_ref: (idx_ref[b], 0, 0)),
            pl.BlockSpec((1, 1, W), lambda b, idx_ref: (idx_ref[b], 0, 0)),
        ],
        out_specs=[
            pl.BlockSpec((1, 1, W2), lambda b, idx_ref: (b, 0, 0)),
            pl.BlockSpec((1, 1, W2), lambda b, idx_ref: (b, 0, 0)),
        ],
    )
    hp2, hc2 = pl.pallas_call(
        _gather_body,
        grid_spec=grid_spec,
        out_shape=[
            jax.ShapeDtypeStruct((B, 1, W2), jnp.float32),
            jax.ShapeDtypeStruct((B, 1, W2), jnp.float32),
        ],
    )(idx, plus.reshape(NW, 1, W), cross.reshape(NW, 1, W))
    return hp2.reshape(B, W2), hc2.reshape(B, W2)


def _params_body(idx_ref, off_ref, s_ref, t_ref, v_ref, fpc_ref, istart_ref):
    iota = jax.lax.broadcasted_iota(jnp.int32, (B, NW), 1)
    onehot = (iota == idx_ref[...]).astype(jnp.float32)
    gathered = jnp.dot(onehot, s_ref[...],
                       preferred_element_type=jnp.float32)
    dec_s = gathered[:, 0:1]
    psi_s = gathered[:, 1:2]
    phi_s = gathered[:, 2:3]
    snr_s = gathered[:, 3:4]

    theta = jnp.pi / 2.0 - dec_s
    ct, st = jnp.cos(theta), jnp.sin(theta)
    cphi, sphi = jnp.cos(phi_s), jnp.sin(phi_s)
    cpsi, spsi = jnp.cos(psi_s), jnp.sin(psi_s)

    u = [ct * cphi, ct * sphi, -st]
    v = [-sphi, cphi, jnp.zeros_like(sphi)]
    mm = [-u[j] * spsi - v[j] * cpsi for j in range(3)]
    nn = [-u[j] * cpsi + v[j] * spsi for j in range(3)]

    fcols = []
    for i in range(NI):
        fp = jnp.zeros_like(dec_s)
        for j in range(3):
            for k in range(3):
                t = t_ref[i, 3 * j + k]
                fp = fp + (mm[j] * mm[k] - nn[j] * nn[k]) * t
        fcols.append(fp)
    for i in range(NI):
        fc = jnp.zeros_like(dec_s)
        for j in range(3):
            for k in range(3):
                t = t_ref[i, 3 * j + k]
                fc = fc + (mm[j] * nn[k] + nn[j] * mm[k]) * t
        fcols.append(fc)

    omega = [st * cphi, st * sphi, ct]
    scols = []
    for i in range(NI):
        dot = (omega[0] * v_ref[i, 0] + omega[1] * v_ref[i, 1]
               + omega[2] * v_ref[i, 2])
        dt = -dot / C_LIGHT
        shift = jnp.round(dt * SAMPLE_RATE).astype(jnp.int32)
        start = jnp.remainder(off_ref[...] - shift, W)
        scols.append(start)

    zf = jnp.zeros_like(dec_s)
    fpc_ref[...] = jnp.concatenate(fcols + [snr_s, zf, zf, zf], axis=1)
    zi = jnp.zeros_like(scols[0])
    istart_ref[...] = jnp.concatenate(
        scols + [zi, zi, zi, zi, zi, zi], axis=1)


def _params(idx_col, off_col, svals, tensors2, vertices):
    return pl.pallas_call(
        _params_body,
        in_specs=[
            pl.BlockSpec((B, 1), lambda: (0, 0)),
            pl.BlockSpec((B, 1), lambda: (0, 0)),
            pl.BlockSpec((NW, 4), lambda: (0, 0)),
            pl.BlockSpec(memory_space=pltpu.SMEM),
            pl.BlockSpec(memory_space=pltpu.SMEM),
        ],
        out_specs=[
            pl.BlockSpec((B, 8), lambda: (0, 0)),
            pl.BlockSpec((B, 8), lambda: (0, 0)),
        ],
        out_shape=[
            jax.ShapeDtypeStruct((B, 8), jnp.float32),
            jax.ShapeDtypeStruct((B, 8), jnp.int32),
        ],
    )(idx_col, off_col, svals, tensors2, vertices)


_BK = 512
_BN = 512
_NTK = W // _BK
_NTN = NF // _BN


def _dft_body(hp_ref, hc_ref, cos_ref, sin_ref, psdT_ref, fpc_ref,
              abc_ref, g_ref, apr, api, acr, aci):
    n = pl.program_id(0)
    k = pl.program_id(1)
    c = cos_ref[...]
    s = sin_ref[...]
    hp = hp_ref[...]
    hc = hc_ref[...]
    pr = jnp.dot(hp, c, preferred_element_type=jnp.float32)
    pi = jnp.dot(hp, s, preferred_element_type=jnp.float32)
    cr = jnp.dot(hc, c, preferred_element_type=jnp.float32)
    ci = jnp.dot(hc, s, preferred_element_type=jnp.float32)

    @pl.when(k == 0)
    def _():
        apr[...] = pr
        api[...] = pi
        acr[...] = cr
        aci[...] = ci

    @pl.when(k > 0)
    def _():
        apr[...] += pr
        api[...] += pi
        acr[...] += cr
        aci[...] += ci

    @pl.when(k == _NTK - 1)
    def _():
        hpr, hpi = apr[...], api[...]
        hcr, hci = acr[...], aci[...]
        P = hpr * hpr + hpi * hpi
        Q = hcr * hcr + hci * hci
        R = hpr * hcr + hpi * hci
        jbin = jax.lax.broadcasted_iota(jnp.int32, (_BN, 1), 0) + n * _BN
        mask = (jbin <= (NBIN_MAX - F0)).astype(jnp.float32)
        wcol = 2.0 * mask / psdT_ref[...]
        a2 = jnp.dot(P, wcol, preferred_element_type=jnp.float32)
        b2 = jnp.dot(Q, wcol, preferred_element_type=jnp.float32)
        c2 = jnp.dot(R, wcol, preferred_element_type=jnp.float32)
        z2 = jnp.zeros_like(a2)
        tile = jnp.concatenate([a2, b2, c2, z2], axis=1)

        @pl.when(n == 0)
        def _():
            abc_ref[...] = tile

        @pl.when(n > 0)
        def _():
            abc_ref[...] += tile

        @pl.when(n == _NTN - 1)
        def _():
            abc = abc_ref[...]
            fpc = fpc_ref[...]
            a = abc[:, 0:2]
            b = abc[:, 2:4]
            cc = abc[:, 4:6]
            Fp = fpc[:, 0:2]
            Fc = fpc[:, 2:4]
            snr_col = fpc[:, 4:5]
            ssq = Fp * Fp * a + Fc * Fc * b + 2.0 * Fp * Fc * cc
            net = jnp.sqrt(ssq[:, 0:1] + ssq[:, 1:2])
            scale = snr_col / net
            zg = jnp.zeros((B, 4), jnp.float32)
            g_ref[...] = jnp.concatenate([scale * Fp, scale * Fc, zg],
                                         axis=1)


def _dft_scales(hp2, hc2, cosm, sinm, psdT, fpc):
    return pl.pallas_call(
        _dft_body,
        grid=(_NTN, _NTK),
        in_specs=[
            pl.BlockSpec((B, _BK), lambda n, k: (0, k)),
            pl.BlockSpec((B, _BK), lambda n, k: (0, k)),
            pl.BlockSpec((_BK, _BN), lambda n, k: (k, n)),
            pl.BlockSpec((_BK, _BN), lambda n, k: (k, n)),
            pl.BlockSpec((_BN, 2), lambda n, k: (n, 0)),
            pl.BlockSpec((B, 8), lambda n, k: (0, 0)),
        ],
        out_specs=[
            pl.BlockSpec((B, 8), lambda n, k: (0, 0)),
            pl.BlockSpec((B, 8), lambda n, k: (0, 0)),
        ],
        out_shape=[
            jax.ShapeDtypeStruct((B, 8), jnp.float32),
            jax.ShapeDtypeStruct((B, 8), jnp.float32),
        ],
        scratch_shapes=[pltpu.VMEM((B, _BN), jnp.float32)] * 4,
    )(hp2, hc2, cosm, sinm, psdT, fpc)


_GB = 8


def _inject_body(hp2_ref, hc2_ref, x_ref, g_ref, s_ref, out_ref):
    p = pl.program_id(0)
    for r in range(_GB):
        b = p * _GB + r
        for i in range(NI):
            s = s_ref[b, i]
            gp = g_ref[b, i]
            gc = g_ref[b, 2 + i]
            row_hp = hp2_ref[pl.ds(r, 1), :W]
            row_hc = hc2_ref[pl.ds(r, 1), :W]
            win_hp = pltpu.roll(row_hp, -s, 1)[0, :KS]
            win_hc = pltpu.roll(row_hc, -s, 1)[0, :KS]
            out_ref[r, i, :] = x_ref[r, i, :] + gp * win_hp + gc * win_hc


def _inject(hp2, hc2, X, g, istart):
    return pl.pallas_call(
        _inject_body,
        grid=(B // _GB,),
        in_specs=[
            pl.BlockSpec((_GB, W2), lambda p: (p, 0)),
            pl.BlockSpec((_GB, W2), lambda p: (p, 0)),
            pl.BlockSpec((_GB, NI, KS), lambda p: (p, 0, 0)),
            pl.BlockSpec(memory_space=pltpu.SMEM),
            pl.BlockSpec(memory_space=pltpu.SMEM),
        ],
        out_specs=pl.BlockSpec((_GB, NI, KS), lambda p: (p, 0, 0)),
        out_shape=jax.ShapeDtypeStruct((B, NI, KS), jnp.float32),
    )(hp2, hc2, X, g, istart)


def kernel(X, plus, cross, dec, psi, phi, snr, tensors, vertices, psd,
           idx, offsets):
    idx = idx.astype(jnp.int32)
    offsets = offsets.astype(jnp.int32)

    hp2, hc2 = _gather_rows(idx, plus, cross)

    svals = jnp.stack([dec, psi, phi, snr], axis=1)
    fpc, istart = _params(idx.reshape(B, 1), offsets.reshape(B, 1),
                          svals, tensors.reshape(NI, 9), vertices)

    cosm = jnp.asarray(_COS)
    sinm = jnp.asarray(_SIN)
    psdT = jnp.concatenate(
        [psd[:, F0:NBIN_MAX + 1],
         jnp.ones((NI, NF - (NBIN_MAX + 1 - F0)), jnp.float32)], axis=1).T
    abc, g = _dft_scales(hp2, hc2, cosm, sinm, psdT, fpc)
    del abc

    return _inject(hp2, hc2, X, g, istart)
